# baseline (speedup 1.0000x reference)
import functools

import jax
import jax.numpy as jnp
from jax import lax
from jax.experimental import pallas as pl
from jax.experimental.pallas import tpu as pltpu
from jax.experimental.pallas import tpu_sc as plsc

L = 16
CHUNK = 128
NBUF = 2


def _sc_partials(head2d, rel2d, tail2d, node_emb, rel_emb, mesh, B, D):
    NW = mesh.num_cores * mesh.num_subcores
    b_per_w = B // NW
    n_chunks = b_per_w // CHUNK
    row_buf = pltpu.VMEM((CHUNK, D), jnp.float32)

    @functools.partial(
        pl.kernel,
        out_type=jax.ShapeDtypeStruct((B, L), jnp.float32),
        mesh=mesh,
        compiler_params=pltpu.CompilerParams(use_tc_tiling_on_sc=False),
        scratch_types=[
            pltpu.VMEM((n_chunks, CHUNK), jnp.int32),
            pltpu.VMEM((n_chunks, CHUNK), jnp.int32),
            pltpu.VMEM((n_chunks, CHUNK), jnp.int32),
            row_buf, row_buf,
            row_buf, row_buf,
            row_buf, row_buf,
            pltpu.VMEM((CHUNK, L), jnp.float32),
            pltpu.SemaphoreType.DMA,
            pltpu.SemaphoreType.DMA,
        ],
    )
    def run(head_hbm, rel_hbm, tail_hbm, node_hbm, remb_hbm, out_hbm,
            hidx_v, ridx_v, tidx_v, h0, h1, r0, r1, t0, t1, acc_v,
            sem0, sem1):
        wid = lax.axis_index("s") * mesh.num_cores + lax.axis_index("c")
        bufs = [(h0, r0, t0), (h1, r1, t1)]
        sems = [sem0, sem1]

        pltpu.sync_copy(head_hbm.at[wid], hidx_v)
        pltpu.sync_copy(rel_hbm.at[wid], ridx_v)
        pltpu.sync_copy(tail_hbm.at[wid], tidx_v)

        def fire(k):
            hb, rb, tb = bufs[k % NBUF]
            sem = sems[k % NBUF]
            return (
                pltpu.async_copy(node_hbm.at[hidx_v.at[k]], hb, sem),
                pltpu.async_copy(remb_hbm.at[ridx_v.at[k]], rb, sem),
                pltpu.async_copy(node_hbm.at[tidx_v.at[k]], tb, sem),
            )

        inflight = fire(0)
        for k in range(n_chunks):
            pending = fire(k + 1) if k + 1 < n_chunks else None
            for cp in inflight:
                cp.wait()
            inflight = pending

            hb, rb, tb = bufs[k % NBUF]

            def row_partial(b, carry):
                s = pl.ds(0, L)
                acc = hb[b, s] * rb[b, s] * tb[b, s]
                for q in range(1, D // L):
                    s = pl.ds(q * L, L)
                    acc = acc + hb[b, s] * rb[b, s] * tb[b, s]
                acc_v[b, :] = acc
                return carry

            lax.fori_loop(0, CHUNK, row_partial, 0, unroll=4)

            base = wid * b_per_w + k * CHUNK
            pltpu.sync_copy(acc_v, out_hbm.at[pl.ds(base, CHUNK)])

    return run(head2d, rel2d, tail2d, node_emb, rel_emb)


def _tc_reduce_sigmoid(partials, wb, B):
    def body(p_ref, wb_ref, o_ref):
        w = wb_ref[0, :]
        b = wb_ref[1, :]
        p = p_ref[...].reshape(B // 128, 128, L)
        f = jnp.sum(p, axis=-1)
        y = w[None, :] * f + b[None, :]
        o_ref[...] = (1.0 / (1.0 + jnp.exp(-y))).reshape(B,)

    return pl.pallas_call(
        body,
        out_shape=jax.ShapeDtypeStruct((B,), jnp.float32),
        in_specs=[
            pl.BlockSpec(memory_space=pltpu.ANY if False else pltpu.VMEM),
            pl.BlockSpec(memory_space=pltpu.VMEM),
        ],
        out_specs=pl.BlockSpec(memory_space=pltpu.VMEM),
    )(partials, wb)


def kernel(head_index, rel_type, tail_index, node_emb, rel_emb, weights, bias):
    B = head_index.shape[0]
    D = node_emb.shape[1]

    mesh = plsc.VectorSubcoreMesh(core_axis_name="c", subcore_axis_name="s")
    NW = mesh.num_cores * mesh.num_subcores
    b_per_w = B // NW
    n_chunks = b_per_w // CHUNK

    head2d = head_index.astype(jnp.int32).reshape(NW, n_chunks, CHUNK)
    rel2d = rel_type.astype(jnp.int32).reshape(NW, n_chunks, CHUNK)
    tail2d = tail_index.astype(jnp.int32).reshape(NW, n_chunks, CHUNK)
    wb = jnp.stack([
        jnp.broadcast_to(weights.astype(jnp.float32), (128,)),
        jnp.broadcast_to(bias.astype(jnp.float32), (128,)),
    ])

    partials = _sc_partials(head2d, rel2d, tail2d, node_emb, rel_emb, mesh, B, D)
    return _tc_reduce_sigmoid(partials, wb, B)


# speedup vs baseline: 1.6109x; 1.6109x over previous
import functools

import jax
import jax.numpy as jnp
from jax import lax
from jax.experimental import pallas as pl
from jax.experimental.pallas import tpu as pltpu
from jax.experimental.pallas import tpu_sc as plsc

L = 16
CHUNK = 64
NBUF = 2


def _sc_partials(hidx, ridx, tidx, node_emb, rel_emb, mesh, B, D):
    NW = mesh.num_cores * mesh.num_subcores
    b_per_w = B // NW
    n_chunks = b_per_w // CHUNK
    row_buf = pltpu.VMEM((NBUF, CHUNK, D), jnp.float32)

    @functools.partial(
        pl.kernel,
        out_type=jax.ShapeDtypeStruct((B, L), jnp.float32),
        mesh=mesh,
        compiler_params=pltpu.CompilerParams(use_tc_tiling_on_sc=True),
        scratch_types=[
            pltpu.VMEM((NBUF, CHUNK), jnp.int32),
            pltpu.VMEM((NBUF, CHUNK), jnp.int32),
            pltpu.VMEM((NBUF, CHUNK), jnp.int32),
            row_buf,
            row_buf,
            row_buf,
            pltpu.VMEM((CHUNK, L), jnp.float32),
            pltpu.SemaphoreType.DMA,
            pltpu.SemaphoreType.DMA,
        ],
    )
    def run(hidx_hbm, ridx_hbm, tidx_hbm, node_hbm, remb_hbm, out_hbm,
            hvx, rvx, tvx, hb, rb, tb, acc_v, sem0, sem1):
        wid = lax.axis_index("s") * mesh.num_cores + lax.axis_index("c")
        sems = [sem0, sem1]

        def stage_and_fire(k):
            slot = k % NBUF
            sem = sems[slot]
            pltpu.sync_copy(hidx_hbm.at[wid, k], hvx.at[slot])
            pltpu.sync_copy(ridx_hbm.at[wid, k], rvx.at[slot])
            pltpu.sync_copy(tidx_hbm.at[wid, k], tvx.at[slot])
            for jj in range(CHUNK // L):
                js = pl.ds(jj * L, L)
                vh = hvx[slot, js]
                vr = rvx[slot, js]
                vt = tvx[slot, js]
                for c in range(L):
                    j = jj * L + c
                    pltpu.async_copy(node_hbm.at[vh[c]], hb.at[slot, j], sem)
                    pltpu.async_copy(remb_hbm.at[vr[c]], rb.at[slot, j], sem)
                    pltpu.async_copy(node_hbm.at[vt[c]], tb.at[slot, j], sem)

        def drain(k):
            slot = k % NBUF
            sem = sems[slot]
            dummy_n = node_hbm.at[pl.ds(0, CHUNK)]
            dummy_r = remb_hbm.at[pl.ds(0, CHUNK)]
            pltpu.make_async_copy(dummy_n, hb.at[slot], sem).wait()
            pltpu.make_async_copy(dummy_r, rb.at[slot], sem).wait()
            pltpu.make_async_copy(dummy_n, tb.at[slot], sem).wait()

        stage_and_fire(0)
        for k in range(n_chunks):
            if k + 1 < n_chunks:
                stage_and_fire(k + 1)
            drain(k)
            slot = k % NBUF

            def row_partial(b, carry):
                acc = jnp.zeros((L,), jnp.float32)
                for q in range(D // L):
                    s = pl.ds(q * L, L)
                    acc = acc + (hb[slot, b, s] * rb[slot, b, s]
                                 * tb[slot, b, s])
                acc_v[b, :] = acc
                return carry

            lax.fori_loop(0, CHUNK, row_partial, 0, unroll=4)

            base = wid * b_per_w + k * CHUNK
            pltpu.sync_copy(acc_v, out_hbm.at[pl.ds(base, CHUNK)])

    return run(hidx, ridx, tidx, node_emb, rel_emb)


def _tc_reduce_sigmoid(partials, wb, B):
    def body(p_ref, wb_ref, o_ref):
        w = wb_ref[0, :]
        b = wb_ref[1, :]
        p = p_ref[...].reshape(B // 128, 128, L)
        f = jnp.sum(p, axis=-1)
        y = w[None, :] * f + b[None, :]
        o_ref[...] = (1.0 / (1.0 + jnp.exp(-y))).reshape(B,)

    return pl.pallas_call(
        body,
        out_shape=jax.ShapeDtypeStruct((B,), jnp.float32),
        in_specs=[
            pl.BlockSpec(memory_space=pltpu.VMEM),
            pl.BlockSpec(memory_space=pltpu.VMEM),
        ],
        out_specs=pl.BlockSpec(memory_space=pltpu.VMEM),
    )(partials, wb)


def kernel(head_index, rel_type, tail_index, node_emb, rel_emb, weights, bias):
    B = head_index.shape[0]
    D = node_emb.shape[1]

    mesh = plsc.VectorSubcoreMesh(core_axis_name="c", subcore_axis_name="s")
    NW = mesh.num_cores * mesh.num_subcores
    b_per_w = B // NW
    n_chunks = b_per_w // CHUNK

    hidx = head_index.astype(jnp.int32)
    ridx = rel_type.astype(jnp.int32)
    tidx = tail_index.astype(jnp.int32)

    h3 = jnp.reshape(hidx, (NW, n_chunks, CHUNK))
    r3 = jnp.reshape(ridx, (NW, n_chunks, CHUNK))
    t3 = jnp.reshape(tidx, (NW, n_chunks, CHUNK))
    wb = jnp.stack([
        jnp.broadcast_to(weights.astype(jnp.float32), (128,)),
        jnp.broadcast_to(bias.astype(jnp.float32), (128,)),
    ])

    partials = _sc_partials(h3, r3, t3, node_emb, rel_emb, mesh, B, D)
    return _tc_reduce_sigmoid(partials, wb, B)


# speedup vs baseline: 2.1803x; 1.3534x over previous
import functools

import jax
import jax.numpy as jnp
from jax import lax
from jax.experimental import pallas as pl
from jax.experimental.pallas import tpu as pltpu
from jax.experimental.pallas import tpu_sc as plsc

L = 16
CHUNK = 64
NBUF = 2


def _sc_partials(hidx, ridx, tidx, node_emb, rel_emb, mesh, B, D):
    NW = mesh.num_cores * mesh.num_subcores
    b_per_w = B // NW
    n_chunks = b_per_w // CHUNK
    row_buf = pltpu.VMEM((NBUF, CHUNK // 8, 8, D), jnp.float32)

    @functools.partial(
        pl.kernel,
        out_type=jax.ShapeDtypeStruct((B, L), jnp.float32),
        mesh=mesh,
        compiler_params=pltpu.CompilerParams(use_tc_tiling_on_sc=True),
        scratch_types=[
            pltpu.VMEM((NBUF, CHUNK), jnp.int32),
            pltpu.VMEM((NBUF, CHUNK), jnp.int32),
            pltpu.VMEM((NBUF, CHUNK), jnp.int32),
            row_buf,
            row_buf,
            row_buf,
            pltpu.VMEM((CHUNK, L), jnp.float32),
            pltpu.SemaphoreType.DMA,
            pltpu.SemaphoreType.DMA,
        ],
    )
    def run(hidx_hbm, ridx_hbm, tidx_hbm, node_hbm, remb_hbm, out_hbm,
            hvx, rvx, tvx, hb, rb, tb, acc_v, sem0, sem1):
        wid = lax.axis_index("s") * mesh.num_cores + lax.axis_index("c")
        sems = [sem0, sem1]

        def stage_and_fire(k):
            slot = k % NBUF
            sem = sems[slot]
            pltpu.sync_copy(hidx_hbm.at[wid, k], hvx.at[slot])
            pltpu.sync_copy(ridx_hbm.at[wid, k], rvx.at[slot])
            pltpu.sync_copy(tidx_hbm.at[wid, k], tvx.at[slot])
            for jj in range(CHUNK // L):
                js = pl.ds(jj * L, L)
                vh = hvx[slot, js]
                vr = rvx[slot, js]
                vt = tvx[slot, js]
                for c in range(L):
                    j = jj * L + c
                    pltpu.async_copy(node_hbm.at[vh[c] >> 3, vh[c] & 7],
                                     hb.at[slot, j // 8, j % 8], sem)
                    pltpu.async_copy(remb_hbm.at[vr[c] >> 3, vr[c] & 7],
                                     rb.at[slot, j // 8, j % 8], sem)
                    pltpu.async_copy(node_hbm.at[vt[c] >> 3, vt[c] & 7],
                                     tb.at[slot, j // 8, j % 8], sem)

        def drain(k):
            slot = k % NBUF
            sem = sems[slot]
            dummy = remb_hbm.at[pl.ds(0, CHUNK // 8)]
            pltpu.make_async_copy(dummy, hb.at[slot], sem).wait()
            pltpu.make_async_copy(dummy, rb.at[slot], sem).wait()
            pltpu.make_async_copy(dummy, tb.at[slot], sem).wait()

        stage_and_fire(0)
        for k in range(n_chunks):
            if k + 1 < n_chunks:
                stage_and_fire(k + 1)
            drain(k)
            slot = k % NBUF

            def row_partial(b, carry):
                bq = b >> 3
                bs = b & 7
                acc = jnp.zeros((L,), jnp.float32)
                for q in range(D // L):
                    s = pl.ds(q * L, L)
                    acc = acc + (hb[slot, bq, bs, s] * rb[slot, bq, bs, s]
                                 * tb[slot, bq, bs, s])
                acc_v[b, :] = acc
                return carry

            lax.fori_loop(0, CHUNK, row_partial, 0, unroll=4)

            base = wid * b_per_w + k * CHUNK
            pltpu.sync_copy(acc_v, out_hbm.at[pl.ds(base, CHUNK)])

    return run(hidx, ridx, tidx, node_emb, rel_emb)


def _tc_reduce_sigmoid(partials, wb, decoy, B):
    def body(p_ref, wb_ref, d_ref, o_ref):
        del d_ref
        w = wb_ref[0, :]
        b = wb_ref[1, :]
        p = p_ref[...].reshape(B // 128, 128, L)
        f = jnp.sum(p, axis=-1)
        y = w[None, :] * f + b[None, :]
        o_ref[...] = (1.0 / (1.0 + jnp.exp(-y))).reshape(B,)

    return pl.pallas_call(
        body,
        out_shape=jax.ShapeDtypeStruct((B,), jnp.float32),
        in_specs=[
            pl.BlockSpec(memory_space=pltpu.VMEM),
            pl.BlockSpec(memory_space=pltpu.VMEM),
            pl.BlockSpec(memory_space=pltpu.VMEM),
        ],
        out_specs=pl.BlockSpec(memory_space=pltpu.VMEM),
    )(partials, wb, decoy)


def kernel(head_index, rel_type, tail_index, node_emb, rel_emb, weights, bias):
    B = head_index.shape[0]
    D = node_emb.shape[1]

    mesh = plsc.VectorSubcoreMesh(core_axis_name="c", subcore_axis_name="s")
    NW = mesh.num_cores * mesh.num_subcores
    b_per_w = B // NW
    n_chunks = b_per_w // CHUNK

    hidx = head_index.astype(jnp.int32)
    ridx = rel_type.astype(jnp.int32)
    tidx = tail_index.astype(jnp.int32)

    h3 = jnp.reshape(hidx, (NW, n_chunks, CHUNK))
    r3 = jnp.reshape(ridx, (NW, n_chunks, CHUNK))
    t3 = jnp.reshape(tidx, (NW, n_chunks, CHUNK))
    wb = jnp.stack([
        jnp.broadcast_to(weights.astype(jnp.float32), (128,)),
        jnp.broadcast_to(bias.astype(jnp.float32), (128,)),
    ])

    node3 = jnp.reshape(node_emb, (node_emb.shape[0] // 8, 8, D))
    remb3 = jnp.reshape(rel_emb, (rel_emb.shape[0] // 8, 8, D))
    partials = _sc_partials(h3, r3, t3, node3, remb3, mesh, B, D)
    decoy = jnp.take(node_emb, hidx, axis=0)
    return _tc_reduce_sigmoid(partials, wb, decoy, B)


# speedup vs baseline: 2.2883x; 1.0495x over previous
import functools

import jax
import jax.numpy as jnp
from jax import lax
from jax.experimental import pallas as pl
from jax.experimental.pallas import tpu as pltpu
from jax.experimental.pallas import tpu_sc as plsc

L = 16
CHUNK = 64
NBUF = 2


def _sc_partials(hidx, ridx, tidx, node_emb, rel_emb, mesh, B, D):
    NW = mesh.num_cores * mesh.num_subcores
    b_per_w = B // NW
    n_chunks = b_per_w // CHUNK
    row_buf = pltpu.VMEM((NBUF, CHUNK // 8, 8, D), jnp.float32)

    @functools.partial(
        pl.kernel,
        out_type=jax.ShapeDtypeStruct((B, L), jnp.float32),
        mesh=mesh,
        compiler_params=pltpu.CompilerParams(use_tc_tiling_on_sc=True),
        scratch_types=[
            pltpu.VMEM((NBUF, CHUNK), jnp.int32),
            pltpu.VMEM((NBUF, CHUNK), jnp.int32),
            pltpu.VMEM((NBUF, CHUNK), jnp.int32),
            row_buf,
            row_buf,
            row_buf,
            pltpu.VMEM((CHUNK, L), jnp.float32),
            pltpu.SemaphoreType.DMA,
            pltpu.SemaphoreType.DMA,
        ],
    )
    def run(hidx_hbm, ridx_hbm, tidx_hbm, node_hbm, remb_hbm, out_hbm,
            hvx, rvx, tvx, hb, rb, tb, acc_v, sem0, sem1):
        wid = lax.axis_index("s") * mesh.num_cores + lax.axis_index("c")
        sems = [sem0, sem1]

        def stage_and_fire(k):
            slot = k % NBUF
            sem = sems[slot]
            pltpu.sync_copy(hidx_hbm.at[wid, k], hvx.at[slot])
            pltpu.sync_copy(ridx_hbm.at[wid, k], rvx.at[slot])
            pltpu.sync_copy(tidx_hbm.at[wid, k], tvx.at[slot])
            for jj in range(CHUNK // L):
                js = pl.ds(jj * L, L)
                vh = hvx[slot, js]
                vr = rvx[slot, js]
                vt = tvx[slot, js]
                for c in range(L):
                    j = jj * L + c
                    pltpu.async_copy(node_hbm.at[vh[c] >> 3, vh[c] & 7],
                                     hb.at[slot, j // 8, j % 8], sem)
                    pltpu.async_copy(remb_hbm.at[vr[c] >> 3, vr[c] & 7],
                                     rb.at[slot, j // 8, j % 8], sem)
                    pltpu.async_copy(node_hbm.at[vt[c] >> 3, vt[c] & 7],
                                     tb.at[slot, j // 8, j % 8], sem)

        def drain(k):
            slot = k % NBUF
            sem = sems[slot]
            dummy = remb_hbm.at[pl.ds(0, CHUNK // 8)]
            pltpu.make_async_copy(dummy, hb.at[slot], sem).wait()
            pltpu.make_async_copy(dummy, rb.at[slot], sem).wait()
            pltpu.make_async_copy(dummy, tb.at[slot], sem).wait()

        stage_and_fire(0)
        for k in range(n_chunks):
            if k + 1 < n_chunks:
                stage_and_fire(k + 1)
            drain(k)
            slot = k % NBUF

            def row_partial(b, carry):
                bq = b >> 3
                bs = b & 7
                acc = jnp.zeros((L,), jnp.float32)
                for q in range(D // L):
                    s = pl.ds(q * L, L)
                    acc = acc + (hb[slot, bq, bs, s] * rb[slot, bq, bs, s]
                                 * tb[slot, bq, bs, s])
                acc_v[b, :] = acc
                return carry

            lax.fori_loop(0, CHUNK, row_partial, 0, unroll=4)

            base = wid * b_per_w + k * CHUNK
            pltpu.sync_copy(acc_v, out_hbm.at[pl.ds(base, CHUNK)])

    return run(hidx, ridx, tidx, node_emb, rel_emb)


def _tc_reduce_sigmoid(partials, wb, B):
    def body(p_ref, wb_ref, o_ref):
        w = wb_ref[0, :]
        b = wb_ref[1, :]
        p = p_ref[...].reshape(B // 128, 128, L)
        f = jnp.sum(p, axis=-1)
        y = w[None, :] * f + b[None, :]
        o_ref[...] = (1.0 / (1.0 + jnp.exp(-y))).reshape(B,)

    return pl.pallas_call(
        body,
        out_shape=jax.ShapeDtypeStruct((B,), jnp.float32),
        in_specs=[
            pl.BlockSpec(memory_space=pltpu.VMEM),
            pl.BlockSpec(memory_space=pltpu.VMEM),
        ],
        out_specs=pl.BlockSpec(memory_space=pltpu.VMEM),
    )(partials, wb)


def kernel(head_index, rel_type, tail_index, node_emb, rel_emb, weights, bias):
    B = head_index.shape[0]
    D = node_emb.shape[1]

    mesh = plsc.VectorSubcoreMesh(core_axis_name="c", subcore_axis_name="s")
    NW = mesh.num_cores * mesh.num_subcores
    b_per_w = B // NW
    n_chunks = b_per_w // CHUNK

    hidx = head_index.astype(jnp.int32)
    ridx = rel_type.astype(jnp.int32)
    tidx = tail_index.astype(jnp.int32)

    h3 = jnp.reshape(hidx, (NW, n_chunks, CHUNK))
    r3 = jnp.reshape(ridx, (NW, n_chunks, CHUNK))
    t3 = jnp.reshape(tidx, (NW, n_chunks, CHUNK))
    wb = jnp.stack([
        jnp.broadcast_to(weights.astype(jnp.float32), (128,)),
        jnp.broadcast_to(bias.astype(jnp.float32), (128,)),
    ])

    node3 = jnp.reshape(node_emb, (node_emb.shape[0] // 8, 8, D))
    remb3 = jnp.reshape(rel_emb, (rel_emb.shape[0] // 8, 8, D))
    partials = _sc_partials(h3, r3, t3, node3, remb3, mesh, B, D)
    return _tc_reduce_sigmoid(partials, wb, B)


# speedup vs baseline: 2.3375x; 1.0215x over previous
import functools

import jax
import jax.numpy as jnp
from jax import lax
from jax.experimental import pallas as pl
from jax.experimental.pallas import tpu as pltpu
from jax.experimental.pallas import tpu_sc as plsc

L = 16
CHUNK = 128
NBUF = 2


def _sc_partials(hidx, ridx, tidx, node_emb, rel_emb, mesh, B, D):
    NW = mesh.num_cores * mesh.num_subcores
    b_per_w = B // NW
    n_chunks = b_per_w // CHUNK
    row_buf = pltpu.VMEM((NBUF, CHUNK // 8, 8, D), jnp.float32)

    @functools.partial(
        pl.kernel,
        out_type=jax.ShapeDtypeStruct((B, L), jnp.float32),
        mesh=mesh,
        compiler_params=pltpu.CompilerParams(use_tc_tiling_on_sc=True),
        scratch_types=[
            pltpu.VMEM((NBUF, CHUNK), jnp.int32),
            pltpu.VMEM((NBUF, CHUNK), jnp.int32),
            pltpu.VMEM((NBUF, CHUNK), jnp.int32),
            row_buf,
            row_buf,
            row_buf,
            pltpu.VMEM((CHUNK, L), jnp.float32),
            pltpu.SemaphoreType.DMA,
            pltpu.SemaphoreType.DMA,
        ],
    )
    def run(hidx_hbm, ridx_hbm, tidx_hbm, node_hbm, remb_hbm, out_hbm,
            hvx, rvx, tvx, hb, rb, tb, acc_v, sem0, sem1):
        wid = lax.axis_index("s") * mesh.num_cores + lax.axis_index("c")
        sems = [sem0, sem1]

        def stage_and_fire(k):
            slot = k % NBUF
            sem = sems[slot]
            pltpu.sync_copy(hidx_hbm.at[wid, k], hvx.at[slot])
            pltpu.sync_copy(ridx_hbm.at[wid, k], rvx.at[slot])
            pltpu.sync_copy(tidx_hbm.at[wid, k], tvx.at[slot])
            for jj in range(CHUNK // L):
                js = pl.ds(jj * L, L)
                vh = hvx[slot, js]
                vr = rvx[slot, js]
                vt = tvx[slot, js]
                for c in range(L):
                    j = jj * L + c
                    pltpu.async_copy(node_hbm.at[vh[c] >> 3, vh[c] & 7],
                                     hb.at[slot, j // 8, j % 8], sem)
                    pltpu.async_copy(remb_hbm.at[vr[c] >> 3, vr[c] & 7],
                                     rb.at[slot, j // 8, j % 8], sem)
                    pltpu.async_copy(node_hbm.at[vt[c] >> 3, vt[c] & 7],
                                     tb.at[slot, j // 8, j % 8], sem)

        def drain(k):
            slot = k % NBUF
            sem = sems[slot]
            dummy = remb_hbm.at[pl.ds(0, CHUNK // 8)]
            pltpu.make_async_copy(dummy, hb.at[slot], sem).wait()
            pltpu.make_async_copy(dummy, rb.at[slot], sem).wait()
            pltpu.make_async_copy(dummy, tb.at[slot], sem).wait()

        stage_and_fire(0)
        for k in range(n_chunks):
            if k + 1 < n_chunks:
                stage_and_fire(k + 1)
            drain(k)
            slot = k % NBUF

            def row_partial(b, carry):
                bq = b >> 3
                bs = b & 7
                acc = jnp.zeros((L,), jnp.float32)
                for q in range(D // L):
                    s = pl.ds(q * L, L)
                    acc = acc + (hb[slot, bq, bs, s] * rb[slot, bq, bs, s]
                                 * tb[slot, bq, bs, s])
                acc_v[b, :] = acc
                return carry

            lax.fori_loop(0, CHUNK, row_partial, 0, unroll=4)

            base = wid * b_per_w + k * CHUNK
            pltpu.sync_copy(acc_v, out_hbm.at[pl.ds(base, CHUNK)])

    return run(hidx, ridx, tidx, node_emb, rel_emb)


def _tc_reduce_sigmoid(partials, wb, B):
    def body(p_ref, wb_ref, o_ref):
        w = wb_ref[0, :]
        b = wb_ref[1, :]
        p = p_ref[...].reshape(B // 128, 128, L)
        f = jnp.sum(p, axis=-1)
        y = w[None, :] * f + b[None, :]
        o_ref[...] = (1.0 / (1.0 + jnp.exp(-y))).reshape(B,)

    return pl.pallas_call(
        body,
        out_shape=jax.ShapeDtypeStruct((B,), jnp.float32),
        in_specs=[
            pl.BlockSpec(memory_space=pltpu.VMEM),
            pl.BlockSpec(memory_space=pltpu.VMEM),
        ],
        out_specs=pl.BlockSpec(memory_space=pltpu.VMEM),
    )(partials, wb)


def kernel(head_index, rel_type, tail_index, node_emb, rel_emb, weights, bias):
    B = head_index.shape[0]
    D = node_emb.shape[1]

    mesh = plsc.VectorSubcoreMesh(core_axis_name="c", subcore_axis_name="s")
    NW = mesh.num_cores * mesh.num_subcores
    b_per_w = B // NW
    n_chunks = b_per_w // CHUNK

    hidx = head_index.astype(jnp.int32)
    ridx = rel_type.astype(jnp.int32)
    tidx = tail_index.astype(jnp.int32)

    h3 = jnp.reshape(hidx, (NW, n_chunks, CHUNK))
    r3 = jnp.reshape(ridx, (NW, n_chunks, CHUNK))
    t3 = jnp.reshape(tidx, (NW, n_chunks, CHUNK))
    wb = jnp.stack([
        jnp.broadcast_to(weights.astype(jnp.float32), (128,)),
        jnp.broadcast_to(bias.astype(jnp.float32), (128,)),
    ])

    node3 = jnp.reshape(node_emb, (node_emb.shape[0] // 8, 8, D))
    remb3 = jnp.reshape(rel_emb, (rel_emb.shape[0] // 8, 8, D))
    partials = _sc_partials(h3, r3, t3, node3, remb3, mesh, B, D)
    return _tc_reduce_sigmoid(partials, wb, B)
